# baseline (speedup 1.0000x reference)
import jax
import jax.numpy as jnp
from jax.experimental import pallas as pl
from jax.experimental.pallas import tpu as pltpu

_ROWS = 16384
_COLS = 128
_CHUNK = 1024
_NCHUNKS = _ROWS // _CHUNK


def _copy_body(x_hbm, o_hbm, buf, in_sem, out_sem):
    def in_cp(i):
        return pltpu.make_async_copy(
            x_hbm.at[pl.ds(i * _CHUNK, _CHUNK), :], buf.at[i], in_sem.at[i]
        )

    def out_cp(i):
        return pltpu.make_async_copy(
            buf.at[i], o_hbm.at[pl.ds(i * _CHUNK, _CHUNK), :], out_sem.at[i]
        )

    for i in range(_NCHUNKS):
        in_cp(i).start()
    for i in range(_NCHUNKS):
        in_cp(i).wait()
        out_cp(i).start()
    for i in range(_NCHUNKS):
        out_cp(i).wait()


def kernel(x):
    return pl.pallas_call(
        _copy_body,
        in_specs=[pl.BlockSpec(memory_space=pl.ANY)],
        out_specs=pl.BlockSpec(memory_space=pl.ANY),
        scratch_shapes=[
            pltpu.VMEM((_NCHUNKS, _CHUNK, _COLS), jnp.float32),
            pltpu.SemaphoreType.DMA((_NCHUNKS,)),
            pltpu.SemaphoreType.DMA((_NCHUNKS,)),
        ],
        out_shape=jax.ShapeDtypeStruct(x.shape, x.dtype),
    )(x)


# speedup vs baseline: 1.0526x; 1.0526x over previous
import jax
import jax.numpy as jnp
from jax.experimental import pallas as pl
from jax.experimental.pallas import tpu as pltpu

_ROWS = 16384
_COLS = 128
_CHUNK = 4096
_NCHUNKS = _ROWS // _CHUNK


def _copy_body(x_hbm, o_hbm, buf, in_sem, out_sem):
    def in_cp(i):
        return pltpu.make_async_copy(
            x_hbm.at[pl.ds(i * _CHUNK, _CHUNK), :], buf.at[i], in_sem.at[i]
        )

    def out_cp(i):
        return pltpu.make_async_copy(
            buf.at[i], o_hbm.at[pl.ds(i * _CHUNK, _CHUNK), :], out_sem.at[i]
        )

    for i in range(_NCHUNKS):
        in_cp(i).start()
    for i in range(_NCHUNKS):
        in_cp(i).wait()
        out_cp(i).start()
    for i in range(_NCHUNKS):
        out_cp(i).wait()


def kernel(x):
    return pl.pallas_call(
        _copy_body,
        in_specs=[pl.BlockSpec(memory_space=pl.ANY)],
        out_specs=pl.BlockSpec(memory_space=pl.ANY),
        scratch_shapes=[
            pltpu.VMEM((_NCHUNKS, _CHUNK, _COLS), jnp.float32),
            pltpu.SemaphoreType.DMA((_NCHUNKS,)),
            pltpu.SemaphoreType.DMA((_NCHUNKS,)),
        ],
        out_shape=jax.ShapeDtypeStruct(x.shape, x.dtype),
    )(x)
